# Initial kernel scaffold; baseline (speedup 1.0000x reference)
import jax
import jax.numpy as jnp
from jax.experimental import pallas as pl


def kernel(x, edge_index, edge_attr, ext, W_r, b_r, W_o, b_o, W_s, b_s):
    raise NotImplementedError("write your pallas kernel here")


# speedup vs baseline: 5.3702x; 5.3702x over previous
import functools

import jax
import jax.numpy as jnp
from jax import lax
from jax.experimental import pallas as pl
from jax.experimental.pallas import tpu as pltpu
from jax.experimental.pallas import tpu_sc as plsc

_N = 10000
_NPAD = 10240
_E = 320000
_NW = 32
_EPW = _E // _NW
_K = 80
_CHUNKS = _EPW // _K
_RPT = _NPAD // 16
_ZB = 128

_BN = 2000


def _pre_body(x_ref, ext_ref, wr_ref, wo_ref, ws_ref, z_ref, xd_ref, base_ref):
    ws = ws_ref[...]
    g = jnp.dot(wo_ref[144:272, :], ws, preferred_element_type=jnp.float32)
    wz = jnp.dot(wr_ref[0:128, :], g, preferred_element_type=jnp.float32)
    wxd = jnp.dot(wr_ref[128:256, :], g, preferred_element_type=jnp.float32)
    wbase = jnp.dot(wo_ref[0:128, :], ws, preferred_element_type=jnp.float32)
    wext = jnp.dot(wo_ref[128:144, :], ws, preferred_element_type=jnp.float32)
    xblk = x_ref[...]
    z_ref[...] = jnp.dot(xblk, wz, preferred_element_type=jnp.float32)
    xd_ref[...] = jnp.dot(xblk, wxd, preferred_element_type=jnp.float32)
    base_ref[...] = (
        jnp.dot(xblk, wbase, preferred_element_type=jnp.float32)
        + jnp.dot(ext_ref[...], wext, preferred_element_type=jnp.float32)
    )


def _sc_body(z_hbm, src_hbm, dst_hbm, ea_hbm, s1_out, s2_out, dg_out,
             srcv, dstv, zrows, earows, onesv, zerov, s1acc, s2acc, dgacc, sem):
    cid = lax.axis_index("c")
    sid = lax.axis_index("s")

    def fill(ref, rows, val):
        def body(i, carry):
            ref[i, :] = jnp.full((16,), val, jnp.float32)
            return carry
        lax.fori_loop(0, rows, body, 0)

    fill(zerov, _ZB, 0.0)
    fill(onesv, _K, 1.0)

    row0 = sid * _RPT
    for j in range(_RPT // _ZB):
        dst_slice = pl.ds(row0 + j * _ZB, _ZB)
        pltpu.sync_copy(zerov, s1acc.at[dst_slice])
        pltpu.sync_copy(zerov, s2acc.at[dst_slice])
        pltpu.sync_copy(zerov, dgacc.at[dst_slice])
    plsc.subcore_barrier()

    off0 = (cid * 16 + sid) * _EPW

    def chunk(i, carry):
        base = off0 + i * _K
        pltpu.sync_copy(src_hbm.at[pl.ds(base, _K)], srcv)
        pltpu.sync_copy(dst_hbm.at[pl.ds(base, _K)], dstv)
        pltpu.async_copy(z_hbm.at[srcv], zrows, sem).wait()
        pltpu.sync_copy(ea_hbm.at[pl.ds(base, _K)], earows)
        pltpu.sync_copy(zrows, s1acc.at[dstv], add=True)
        pltpu.sync_copy(earows, s2acc.at[dstv], add=True)
        pltpu.sync_copy(onesv, dgacc.at[dstv], add=True)
        return carry

    lax.fori_loop(0, _CHUNKS, chunk, 0)
    plsc.subcore_barrier()

    out_slice = pl.ds(cid * _NPAD + row0, _RPT)
    acc_slice = pl.ds(row0, _RPT)
    pltpu.sync_copy(s1acc.at[acc_slice], s1_out.at[out_slice])
    pltpu.sync_copy(s2acc.at[acc_slice], s2_out.at[out_slice])
    pltpu.sync_copy(dgacc.at[acc_slice], dg_out.at[out_slice])


_sc_segsum = functools.partial(
    pl.kernel,
    out_type=[jax.ShapeDtypeStruct((2 * _NPAD, 16), jnp.float32)] * 3,
    mesh=plsc.VectorSubcoreMesh(core_axis_name="c", subcore_axis_name="s"),
    scratch_types=[
        pltpu.VMEM((_K,), jnp.int32),
        pltpu.VMEM((_K,), jnp.int32),
        pltpu.VMEM((_K, 16), jnp.float32),
        pltpu.VMEM((_K, 16), jnp.float32),
        pltpu.VMEM((_K, 16), jnp.float32),
        pltpu.VMEM((_ZB, 16), jnp.float32),
        pltpu.VMEM_SHARED((_NPAD, 16), jnp.float32),
        pltpu.VMEM_SHARED((_NPAD, 16), jnp.float32),
        pltpu.VMEM_SHARED((_NPAD, 16), jnp.float32),
        pltpu.SemaphoreType.DMA,
    ],
    compiler_params=pltpu.CompilerParams(use_tc_tiling_on_sc=False),
)(_sc_body)


def _post_body(s1_ref, s2_ref, dg_ref, xd_ref, base_ref,
               wr_ref, wo_ref, ws_ref, br_ref, bo_ref, bs_ref, out_ref):
    ws = ws_ref[...]
    g = jnp.dot(wo_ref[144:272, :], ws, preferred_element_type=jnp.float32)
    wea = jnp.dot(wr_ref[256:272, :], g, preferred_element_type=jnp.float32)
    c16 = jnp.dot(br_ref[...], g, preferred_element_type=jnp.float32)
    cb = jnp.dot(bo_ref[...], ws, preferred_element_type=jnp.float32) + bs_ref[...]

    s1 = s1_ref[0] + s1_ref[1]
    s2 = s2_ref[0] + s2_ref[1]
    deg = dg_ref[0] + dg_ref[1]
    scores = (
        base_ref[...] + s1
        + deg * (xd_ref[...] + c16)
        + jnp.dot(s2, wea, preferred_element_type=jnp.float32)
        + cb
    )
    m = jnp.max(scores, axis=1, keepdims=True)
    e = jnp.exp(scores - m)
    out_ref[...] = e / jnp.sum(e, axis=1, keepdims=True)


def kernel(x, edge_index, edge_attr, ext, W_r, b_r, W_o, b_o, W_s, b_s):
    n, ds = x.shape
    e = edge_index.shape[1]
    assert (n, ds, e) == (_N, 128, _E)

    grid = (_N // _BN,)
    full = lambda shape: pl.BlockSpec(shape, lambda i: (0, 0))
    blk16 = pl.BlockSpec((_BN, 16), lambda i: (i, 0))

    z, xd16, base16 = pl.pallas_call(
        _pre_body,
        grid=grid,
        in_specs=[
            pl.BlockSpec((_BN, 128), lambda i: (i, 0)),
            blk16,
            full((272, 128)),
            full((272, 128)),
            full((128, 16)),
        ],
        out_specs=[blk16, blk16, blk16],
        out_shape=[jax.ShapeDtypeStruct((_N, 16), jnp.float32)] * 3,
    )(x, ext, W_r, W_o, W_s)

    src = edge_index[0]
    dst = edge_index[1]
    s1p, s2p, dgp = _sc_segsum(z, src, dst, edge_attr)
    unpad = lambda a: jnp.stack([a[:_N], a[_NPAD:_NPAD + _N]])
    s1p = unpad(s1p)
    s2p = unpad(s2p)
    dgp = unpad(dgp)

    pblk16 = pl.BlockSpec((2, _BN, 16), lambda i: (0, i, 0))
    probs = pl.pallas_call(
        _post_body,
        grid=grid,
        in_specs=[
            pblk16, pblk16, pblk16, blk16, blk16,
            full((272, 128)),
            full((272, 128)),
            full((128, 16)),
            full((1, 128)),
            full((1, 128)),
            full((1, 16)),
        ],
        out_specs=blk16,
        out_shape=jax.ShapeDtypeStruct((_N, 16), jnp.float32),
    )(s1p, s2p, dgp, xd16, base16, W_r, W_o, W_s,
      b_r.reshape(1, 128), b_o.reshape(1, 128), b_s.reshape(1, 16))
    return probs


# speedup vs baseline: 8.6647x; 1.6135x over previous
import functools

import jax
import jax.numpy as jnp
from jax import lax
from jax.experimental import pallas as pl
from jax.experimental.pallas import tpu as pltpu
from jax.experimental.pallas import tpu_sc as plsc

_N = 10000
_NPAD = 10240
_E = 320000
_NW = 32
_EPW = _E // _NW
_K = 400
_CHUNKS = _EPW // _K
_RPT = _NPAD // 16
_ZB = 128

_BN = 2000


def _pre_body(x_ref, ext_ref, wr_ref, wo_ref, ws_ref, z_ref, xd_ref, base_ref):
    ws = ws_ref[...]
    g = jnp.dot(wo_ref[144:272, :], ws, preferred_element_type=jnp.float32)
    wz = jnp.dot(wr_ref[0:128, :], g, preferred_element_type=jnp.float32)
    wxd = jnp.dot(wr_ref[128:256, :], g, preferred_element_type=jnp.float32)
    wbase = jnp.dot(wo_ref[0:128, :], ws, preferred_element_type=jnp.float32)
    wext = jnp.dot(wo_ref[128:144, :], ws, preferred_element_type=jnp.float32)
    xblk = x_ref[...]
    z_ref[...] = jnp.dot(xblk, wz, preferred_element_type=jnp.float32)
    xd_ref[...] = jnp.dot(xblk, wxd, preferred_element_type=jnp.float32)
    base_ref[...] = (
        jnp.dot(xblk, wbase, preferred_element_type=jnp.float32)
        + jnp.dot(ext_ref[...], wext, preferred_element_type=jnp.float32)
    )


def _sc_body(z_hbm, src_hbm, dst_hbm, ea_hbm, s1_out, s2_out, dg_out,
             srcv, dstv, zrows, earows, onesv, zerov, s1acc, s2acc, dgacc, sem):
    cid = lax.axis_index("c")
    sid = lax.axis_index("s")

    def fill(ref, rows, val):
        def body(i, carry):
            ref[i, :] = jnp.full((16,), val, jnp.float32)
            return carry
        lax.fori_loop(0, rows, body, 0)

    fill(zerov, _ZB, 0.0)
    fill(onesv, _K, 1.0)

    row0 = sid * _RPT
    for j in range(_RPT // _ZB):
        dst_slice = pl.ds(row0 + j * _ZB, _ZB)
        pltpu.sync_copy(zerov, s1acc.at[dst_slice])
        pltpu.sync_copy(zerov, s2acc.at[dst_slice])
        pltpu.sync_copy(zerov, dgacc.at[dst_slice])
    plsc.subcore_barrier()

    off0 = (cid * 16 + sid) * _EPW

    def chunk(i, carry):
        base = off0 + i * _K
        pltpu.sync_copy(src_hbm.at[pl.ds(base, _K)], srcv)
        pltpu.sync_copy(dst_hbm.at[pl.ds(base, _K)], dstv)
        pltpu.async_copy(z_hbm.at[srcv], zrows, sem).wait()
        pltpu.sync_copy(ea_hbm.at[pl.ds(base, _K)], earows)
        pltpu.sync_copy(zrows, s1acc.at[dstv], add=True)
        pltpu.sync_copy(earows, s2acc.at[dstv], add=True)
        pltpu.sync_copy(onesv, dgacc.at[dstv], add=True)
        return carry

    lax.fori_loop(0, _CHUNKS, chunk, 0)
    plsc.subcore_barrier()

    out_slice = pl.ds(cid * _NPAD + row0, _RPT)
    acc_slice = pl.ds(row0, _RPT)
    pltpu.sync_copy(s1acc.at[acc_slice], s1_out.at[out_slice])
    pltpu.sync_copy(s2acc.at[acc_slice], s2_out.at[out_slice])
    pltpu.sync_copy(dgacc.at[acc_slice], dg_out.at[out_slice])


_sc_segsum = functools.partial(
    pl.kernel,
    out_type=[jax.ShapeDtypeStruct((2 * _NPAD, 16), jnp.float32)] * 3,
    mesh=plsc.VectorSubcoreMesh(core_axis_name="c", subcore_axis_name="s"),
    scratch_types=[
        pltpu.VMEM((_K,), jnp.int32),
        pltpu.VMEM((_K,), jnp.int32),
        pltpu.VMEM((_K, 16), jnp.float32),
        pltpu.VMEM((_K, 16), jnp.float32),
        pltpu.VMEM((_K, 16), jnp.float32),
        pltpu.VMEM((_ZB, 16), jnp.float32),
        pltpu.VMEM_SHARED((_NPAD, 16), jnp.float32),
        pltpu.VMEM_SHARED((_NPAD, 16), jnp.float32),
        pltpu.VMEM_SHARED((_NPAD, 16), jnp.float32),
        pltpu.SemaphoreType.DMA,
    ],
    compiler_params=pltpu.CompilerParams(use_tc_tiling_on_sc=False),
)(_sc_body)


def _post_body(s1_ref, s2_ref, dg_ref, xd_ref, base_ref,
               wr_ref, wo_ref, ws_ref, br_ref, bo_ref, bs_ref, out_ref):
    ws = ws_ref[...]
    g = jnp.dot(wo_ref[144:272, :], ws, preferred_element_type=jnp.float32)
    wea = jnp.dot(wr_ref[256:272, :], g, preferred_element_type=jnp.float32)
    c16 = jnp.dot(br_ref[...], g, preferred_element_type=jnp.float32)
    cb = jnp.dot(bo_ref[...], ws, preferred_element_type=jnp.float32) + bs_ref[...]

    s1 = s1_ref[0] + s1_ref[1]
    s2 = s2_ref[0] + s2_ref[1]
    deg = dg_ref[0] + dg_ref[1]
    scores = (
        base_ref[...] + s1
        + deg * (xd_ref[...] + c16)
        + jnp.dot(s2, wea, preferred_element_type=jnp.float32)
        + cb
    )
    m = jnp.max(scores, axis=1, keepdims=True)
    e = jnp.exp(scores - m)
    out_ref[...] = e / jnp.sum(e, axis=1, keepdims=True)


def kernel(x, edge_index, edge_attr, ext, W_r, b_r, W_o, b_o, W_s, b_s):
    n, ds = x.shape
    e = edge_index.shape[1]
    assert (n, ds, e) == (_N, 128, _E)

    grid = (_N // _BN,)
    full = lambda shape: pl.BlockSpec(shape, lambda i: (0, 0))
    blk16 = pl.BlockSpec((_BN, 16), lambda i: (i, 0))

    z, xd16, base16 = pl.pallas_call(
        _pre_body,
        grid=grid,
        in_specs=[
            pl.BlockSpec((_BN, 128), lambda i: (i, 0)),
            blk16,
            full((272, 128)),
            full((272, 128)),
            full((128, 16)),
        ],
        out_specs=[blk16, blk16, blk16],
        out_shape=[jax.ShapeDtypeStruct((_N, 16), jnp.float32)] * 3,
    )(x, ext, W_r, W_o, W_s)

    src = edge_index[0]
    dst = edge_index[1]
    s1p, s2p, dgp = _sc_segsum(z, src, dst, edge_attr)
    unpad = lambda a: jnp.stack([a[:_N], a[_NPAD:_NPAD + _N]])
    s1p = unpad(s1p)
    s2p = unpad(s2p)
    dgp = unpad(dgp)

    pblk16 = pl.BlockSpec((2, _BN, 16), lambda i: (0, i, 0))
    probs = pl.pallas_call(
        _post_body,
        grid=grid,
        in_specs=[
            pblk16, pblk16, pblk16, blk16, blk16,
            full((272, 128)),
            full((272, 128)),
            full((128, 16)),
            full((1, 128)),
            full((1, 128)),
            full((1, 16)),
        ],
        out_specs=blk16,
        out_shape=jax.ShapeDtypeStruct((_N, 16), jnp.float32),
    )(s1p, s2p, dgp, xd16, base16, W_r, W_o, W_s,
      b_r.reshape(1, 128), b_o.reshape(1, 128), b_s.reshape(1, 16))
    return probs


# speedup vs baseline: 9.5256x; 1.0994x over previous
import functools

import jax
import jax.numpy as jnp
from jax import lax
from jax.experimental import pallas as pl
from jax.experimental.pallas import tpu as pltpu
from jax.experimental.pallas import tpu_sc as plsc

_N = 10000
_NPAD = 10240
_E = 320000
_NW = 32
_EPW = _E // _NW
_K = 1000
_CHUNKS = _EPW // _K
_RPT = _NPAD // 16
_ZB = 128

_BN = 2000


def _pre_body(x_ref, ext_ref, wr_ref, wo_ref, ws_ref, z_ref, xd_ref, base_ref):
    ws = ws_ref[...]
    g = jnp.dot(wo_ref[144:272, :], ws, preferred_element_type=jnp.float32)
    wz = jnp.dot(wr_ref[0:128, :], g, preferred_element_type=jnp.float32)
    wxd = jnp.dot(wr_ref[128:256, :], g, preferred_element_type=jnp.float32)
    wbase = jnp.dot(wo_ref[0:128, :], ws, preferred_element_type=jnp.float32)
    wext = jnp.dot(wo_ref[128:144, :], ws, preferred_element_type=jnp.float32)
    xblk = x_ref[...]
    z_ref[...] = jnp.dot(xblk, wz, preferred_element_type=jnp.float32)
    xd_ref[...] = jnp.dot(xblk, wxd, preferred_element_type=jnp.float32)
    base_ref[...] = (
        jnp.dot(xblk, wbase, preferred_element_type=jnp.float32)
        + jnp.dot(ext_ref[...], wext, preferred_element_type=jnp.float32)
    )


def _sc_body(z_hbm, src_hbm, dst_hbm, ea_hbm, s1_out, s2_out, dg_out,
             srcv, dstv, zrows, earows, onesv, zerov, s1acc, s2acc, dgacc, sem):
    cid = lax.axis_index("c")
    sid = lax.axis_index("s")

    def fill(ref, rows, val):
        def body(i, carry):
            ref[i, :] = jnp.full((16,), val, jnp.float32)
            return carry
        lax.fori_loop(0, rows, body, 0)

    fill(zerov, _ZB, 0.0)
    fill(onesv, _K, 1.0)

    row0 = sid * _RPT
    for j in range(_RPT // _ZB):
        dst_slice = pl.ds(row0 + j * _ZB, _ZB)
        pltpu.sync_copy(zerov, s1acc.at[dst_slice])
        pltpu.sync_copy(zerov, s2acc.at[dst_slice])
        pltpu.sync_copy(zerov, dgacc.at[dst_slice])
    plsc.subcore_barrier()

    off0 = (cid * 16 + sid) * _EPW

    def chunk(i, carry):
        base = off0 + i * _K
        pltpu.sync_copy(src_hbm.at[pl.ds(base, _K)], srcv)
        pltpu.sync_copy(dst_hbm.at[pl.ds(base, _K)], dstv)
        pltpu.async_copy(z_hbm.at[srcv], zrows, sem).wait()
        pltpu.sync_copy(ea_hbm.at[pl.ds(base, _K)], earows)
        pltpu.sync_copy(zrows, s1acc.at[dstv], add=True)
        pltpu.sync_copy(earows, s2acc.at[dstv], add=True)
        pltpu.sync_copy(onesv, dgacc.at[dstv], add=True)
        return carry

    lax.fori_loop(0, _CHUNKS, chunk, 0)
    plsc.subcore_barrier()

    out_slice = pl.ds(cid * _NPAD + row0, _RPT)
    acc_slice = pl.ds(row0, _RPT)
    pltpu.sync_copy(s1acc.at[acc_slice], s1_out.at[out_slice])
    pltpu.sync_copy(s2acc.at[acc_slice], s2_out.at[out_slice])
    pltpu.sync_copy(dgacc.at[acc_slice], dg_out.at[out_slice])


_sc_segsum = functools.partial(
    pl.kernel,
    out_type=[jax.ShapeDtypeStruct((2 * _NPAD, 16), jnp.float32)] * 3,
    mesh=plsc.VectorSubcoreMesh(core_axis_name="c", subcore_axis_name="s"),
    scratch_types=[
        pltpu.VMEM((_K,), jnp.int32),
        pltpu.VMEM((_K,), jnp.int32),
        pltpu.VMEM((_K, 16), jnp.float32),
        pltpu.VMEM((_K, 16), jnp.float32),
        pltpu.VMEM((_K, 16), jnp.float32),
        pltpu.VMEM((_ZB, 16), jnp.float32),
        pltpu.VMEM_SHARED((_NPAD, 16), jnp.float32),
        pltpu.VMEM_SHARED((_NPAD, 16), jnp.float32),
        pltpu.VMEM_SHARED((_NPAD, 16), jnp.float32),
        pltpu.SemaphoreType.DMA,
    ],
    compiler_params=pltpu.CompilerParams(use_tc_tiling_on_sc=False),
)(_sc_body)


def _post_body(s1_ref, s2_ref, dg_ref, xd_ref, base_ref,
               wr_ref, wo_ref, ws_ref, br_ref, bo_ref, bs_ref, out_ref):
    ws = ws_ref[...]
    g = jnp.dot(wo_ref[144:272, :], ws, preferred_element_type=jnp.float32)
    wea = jnp.dot(wr_ref[256:272, :], g, preferred_element_type=jnp.float32)
    c16 = jnp.dot(br_ref[...], g, preferred_element_type=jnp.float32)
    cb = jnp.dot(bo_ref[...], ws, preferred_element_type=jnp.float32) + bs_ref[...]

    s1 = s1_ref[0] + s1_ref[1]
    s2 = s2_ref[0] + s2_ref[1]
    deg = dg_ref[0] + dg_ref[1]
    scores = (
        base_ref[...] + s1
        + deg * (xd_ref[...] + c16)
        + jnp.dot(s2, wea, preferred_element_type=jnp.float32)
        + cb
    )
    m = jnp.max(scores, axis=1, keepdims=True)
    e = jnp.exp(scores - m)
    out_ref[...] = e / jnp.sum(e, axis=1, keepdims=True)


def kernel(x, edge_index, edge_attr, ext, W_r, b_r, W_o, b_o, W_s, b_s):
    n, ds = x.shape
    e = edge_index.shape[1]
    assert (n, ds, e) == (_N, 128, _E)

    grid = (_N // _BN,)
    full = lambda shape: pl.BlockSpec(shape, lambda i: (0, 0))
    blk16 = pl.BlockSpec((_BN, 16), lambda i: (i, 0))

    z, xd16, base16 = pl.pallas_call(
        _pre_body,
        grid=grid,
        in_specs=[
            pl.BlockSpec((_BN, 128), lambda i: (i, 0)),
            blk16,
            full((272, 128)),
            full((272, 128)),
            full((128, 16)),
        ],
        out_specs=[blk16, blk16, blk16],
        out_shape=[jax.ShapeDtypeStruct((_N, 16), jnp.float32)] * 3,
    )(x, ext, W_r, W_o, W_s)

    src = edge_index[0]
    dst = edge_index[1]
    s1p, s2p, dgp = _sc_segsum(z, src, dst, edge_attr)
    unpad = lambda a: jnp.stack([a[:_N], a[_NPAD:_NPAD + _N]])
    s1p = unpad(s1p)
    s2p = unpad(s2p)
    dgp = unpad(dgp)

    pblk16 = pl.BlockSpec((2, _BN, 16), lambda i: (0, i, 0))
    probs = pl.pallas_call(
        _post_body,
        grid=grid,
        in_specs=[
            pblk16, pblk16, pblk16, blk16, blk16,
            full((272, 128)),
            full((272, 128)),
            full((128, 16)),
            full((1, 128)),
            full((1, 128)),
            full((1, 16)),
        ],
        out_specs=blk16,
        out_shape=jax.ShapeDtypeStruct((_N, 16), jnp.float32),
    )(s1p, s2p, dgp, xd16, base16, W_r, W_o, W_s,
      b_r.reshape(1, 128), b_o.reshape(1, 128), b_s.reshape(1, 16))
    return probs


# speedup vs baseline: 12.1062x; 1.2709x over previous
import functools

import jax
import jax.numpy as jnp
from jax import lax
from jax.experimental import pallas as pl
from jax.experimental.pallas import tpu as pltpu
from jax.experimental.pallas import tpu_sc as plsc

_N = 10000
_NPAD = 12000
_E = 320000
_NW = 32
_EPW = _E // _NW
_K = 400
_CHUNKS = _EPW // _K
_RPT = _NPAD // 16
_ZB = 250

_BN = 2000


def _pre_body(x_ref, ext_ref, wr_ref, wo_ref, ws_ref, z_ref, xd_ref, base_ref):
    ws = ws_ref[...]
    g = jnp.dot(wo_ref[144:272, :], ws, preferred_element_type=jnp.float32)
    wz = jnp.dot(wr_ref[0:128, :], g, preferred_element_type=jnp.float32)
    wxd = jnp.dot(wr_ref[128:256, :], g, preferred_element_type=jnp.float32)
    wbase = jnp.dot(wo_ref[0:128, :], ws, preferred_element_type=jnp.float32)
    wext = jnp.dot(wo_ref[128:144, :], ws, preferred_element_type=jnp.float32)
    xblk = x_ref[...]
    z_ref[...] = jnp.dot(xblk, wz, preferred_element_type=jnp.float32)
    xd_ref[...] = jnp.dot(xblk, wxd, preferred_element_type=jnp.float32)
    base_ref[...] = (
        jnp.dot(xblk, wbase, preferred_element_type=jnp.float32)
        + jnp.dot(ext_ref[...], wext, preferred_element_type=jnp.float32)
    )


def _sc_body(z_hbm, ei_hbm, ea_hbm, s1_out, s2_out, dg_out,
             srcv0, dstv0, zrows0, earows0, srcv1, dstv1, zrows1, earows1,
             onesv, zerov, s1acc, s2acc, dgacc,
             sem_ld0, sem_g0, sem_sc0, sem_ld1, sem_g1, sem_sc1):
    cid = lax.axis_index("c")
    sid = lax.axis_index("s")
    srcv = (srcv0, srcv1)
    dstv = (dstv0, dstv1)
    zrows = (zrows0, zrows1)
    earows = (earows0, earows1)
    sem_ld = (sem_ld0, sem_ld1)
    sem_g = (sem_g0, sem_g1)
    sem_sc = (sem_sc0, sem_sc1)

    def fill(ref, rows, val):
        def body(i, carry):
            ref[i, :] = jnp.full((16,), val, jnp.float32)
            return carry
        lax.fori_loop(0, rows, body, 0)

    fill(zerov, _ZB, 0.0)
    fill(onesv, _K, 1.0)

    row0 = sid * _RPT
    zs = []
    for j in range(_RPT // _ZB):
        dst_slice = pl.ds(row0 + j * _ZB, _ZB)
        zs.append(pltpu.async_copy(zerov, s1acc.at[dst_slice], sem_ld0))
        zs.append(pltpu.async_copy(zerov, s2acc.at[dst_slice], sem_ld0))
        zs.append(pltpu.async_copy(zerov, dgacc.at[dst_slice], sem_ld0))
    for h in zs:
        h.wait()
    plsc.subcore_barrier()

    off0 = (cid * 16 + sid) * _EPW

    def loads(c, s):
        base = off0 + c * _K
        return (
            pltpu.async_copy(ei_hbm.at[0, pl.ds(base, _K)], srcv[s], sem_ld[s]),
            pltpu.async_copy(ei_hbm.at[1, pl.ds(base, _K)], dstv[s], sem_ld[s]),
            pltpu.async_copy(ea_hbm.at[pl.ds(base, _K)], earows[s], sem_ld[s]),
        )

    ldp = [None, None]
    scp = [None, None]
    ldp[0] = loads(0, 0)
    for i in range(_CHUNKS):
        s = i & 1
        o = 1 - s
        for h in ldp[s]:
            h.wait()
        g = pltpu.async_copy(z_hbm.at[srcv[s]], zrows[s], sem_g[s])
        if scp[o] is not None:
            for h in scp[o]:
                h.wait()
        if i + 1 < _CHUNKS:
            ldp[o] = loads(i + 1, o)
        g.wait()
        scp[s] = (
            pltpu.async_copy(zrows[s], s1acc.at[dstv[s]], sem_sc[s], add=True),
            pltpu.async_copy(earows[s], s2acc.at[dstv[s]], sem_sc[s], add=True),
            pltpu.async_copy(onesv, dgacc.at[dstv[s]], sem_sc[s], add=True),
        )
    for h in scp[(_CHUNKS - 1) & 1]:
        h.wait()
    plsc.subcore_barrier()

    out_slice = pl.ds(cid * _NPAD + row0, _RPT)
    acc_slice = pl.ds(row0, _RPT)
    pltpu.sync_copy(s1acc.at[acc_slice], s1_out.at[out_slice])
    pltpu.sync_copy(s2acc.at[acc_slice], s2_out.at[out_slice])
    pltpu.sync_copy(dgacc.at[acc_slice], dg_out.at[out_slice])


_sc_segsum = functools.partial(
    pl.kernel,
    out_type=[jax.ShapeDtypeStruct((2 * _NPAD, 16), jnp.float32)] * 3,
    mesh=plsc.VectorSubcoreMesh(core_axis_name="c", subcore_axis_name="s"),
    scratch_types=[
        pltpu.VMEM((_K,), jnp.int32),
        pltpu.VMEM((_K,), jnp.int32),
        pltpu.VMEM((_K, 16), jnp.float32),
        pltpu.VMEM((_K, 16), jnp.float32),
        pltpu.VMEM((_K,), jnp.int32),
        pltpu.VMEM((_K,), jnp.int32),
        pltpu.VMEM((_K, 16), jnp.float32),
        pltpu.VMEM((_K, 16), jnp.float32),
        pltpu.VMEM((_K, 16), jnp.float32),
        pltpu.VMEM((_ZB, 16), jnp.float32),
        pltpu.VMEM_SHARED((_NPAD, 16), jnp.float32),
        pltpu.VMEM_SHARED((_NPAD, 16), jnp.float32),
        pltpu.VMEM_SHARED((_NPAD, 16), jnp.float32),
        pltpu.SemaphoreType.DMA,
        pltpu.SemaphoreType.DMA,
        pltpu.SemaphoreType.DMA,
        pltpu.SemaphoreType.DMA,
        pltpu.SemaphoreType.DMA,
        pltpu.SemaphoreType.DMA,
    ],
    compiler_params=pltpu.CompilerParams(use_tc_tiling_on_sc=False),
)(_sc_body)


def _post_body(s1a_ref, s1b_ref, s2a_ref, s2b_ref, dga_ref, dgb_ref,
               xd_ref, base_ref,
               wr_ref, wo_ref, ws_ref, br_ref, bo_ref, bs_ref, out_ref):
    ws = ws_ref[...]
    g = jnp.dot(wo_ref[144:272, :], ws, preferred_element_type=jnp.float32)
    wea = jnp.dot(wr_ref[256:272, :], g, preferred_element_type=jnp.float32)
    c16 = jnp.dot(br_ref[...], g, preferred_element_type=jnp.float32)
    cb = jnp.dot(bo_ref[...], ws, preferred_element_type=jnp.float32) + bs_ref[...]

    s1 = s1a_ref[...] + s1b_ref[...]
    s2 = s2a_ref[...] + s2b_ref[...]
    deg = dga_ref[...] + dgb_ref[...]
    scores = (
        base_ref[...] + s1
        + deg * (xd_ref[...] + c16)
        + jnp.dot(s2, wea, preferred_element_type=jnp.float32)
        + cb
    )
    m = jnp.max(scores, axis=1, keepdims=True)
    e = jnp.exp(scores - m)
    out_ref[...] = e / jnp.sum(e, axis=1, keepdims=True)


def kernel(x, edge_index, edge_attr, ext, W_r, b_r, W_o, b_o, W_s, b_s):
    n, ds = x.shape
    e = edge_index.shape[1]
    assert (n, ds, e) == (_N, 128, _E)

    grid = (_N // _BN,)
    full = lambda shape: pl.BlockSpec(shape, lambda i: (0, 0))
    blk16 = pl.BlockSpec((_BN, 16), lambda i: (i, 0))

    z, xd16, base16 = pl.pallas_call(
        _pre_body,
        grid=grid,
        in_specs=[
            pl.BlockSpec((_BN, 128), lambda i: (i, 0)),
            blk16,
            full((272, 128)),
            full((272, 128)),
            full((128, 16)),
        ],
        out_specs=[blk16, blk16, blk16],
        out_shape=[jax.ShapeDtypeStruct((_N, 16), jnp.float32)] * 3,
    )(x, ext, W_r, W_o, W_s)

    s1p, s2p, dgp = _sc_segsum(z, edge_index, edge_attr)

    lo16 = pl.BlockSpec((_BN, 16), lambda i: (i, 0))
    hi16 = pl.BlockSpec((_BN, 16), lambda i: (i + _NPAD // _BN, 0))
    probs = pl.pallas_call(
        _post_body,
        grid=grid,
        in_specs=[
            lo16, hi16, lo16, hi16, lo16, hi16, blk16, blk16,
            full((272, 128)),
            full((272, 128)),
            full((128, 16)),
            full((1, 128)),
            full((1, 128)),
            full((1, 16)),
        ],
        out_specs=blk16,
        out_shape=jax.ShapeDtypeStruct((_N, 16), jnp.float32),
    )(s1p, s1p, s2p, s2p, dgp, dgp, xd16, base16, W_r, W_o, W_s,
      b_r.reshape(1, 128), b_o.reshape(1, 128), b_s.reshape(1, 16))
    return probs
